# baseline (speedup 1.0000x reference)
import functools

import jax
import jax.numpy as jnp
from jax import lax
from jax.experimental import pallas as pl
from jax.experimental.pallas import tpu as pltpu
from jax.experimental.pallas import tpu_sc as plsc

B = 16384
D = 256
L = 16
NC = 2
NS = 16
NW = NC * NS
BPW = B // NW
Q = 4
NQ = BPW // Q
NPAT = 1 << Q

_mesh = plsc.VectorSubcoreMesh(core_axis_name="c", subcore_axis_name="s")


@functools.partial(
    pl.kernel,
    mesh=_mesh,
    out_type=jax.ShapeDtypeStruct((B, D), jnp.float32),
    scratch_types=[
        pltpu.VMEM((BPW + L,), jnp.int32),
        pltpu.VMEM((NPAT * Q, D), jnp.float32),
        pltpu.SemaphoreType.DMA,
        pltpu.SemaphoreType.DMA,
    ],
)
def _lookup(idx_hbm, table_hbm, out_hbm, idx_v, pat_v, sem, bsem):
    wid = lax.axis_index("s") * NC + lax.axis_index("c")
    base = wid * BPW
    pltpu.sync_copy(idx_hbm.at[wid], idx_v.at[pl.ds(0, BPW)])

    def build(k, carry):
        h = k & (Q - 1)
        p = k >> 2
        bit = lax.shift_right_logical(p, (Q - 1) - h) & 1
        pltpu.async_copy(table_hbm.at[pl.ds(bit, 1)],
                         pat_v.at[pl.ds(k, 1)], bsem)
        return carry
    lax.fori_loop(0, NPAT * Q, build, 0)
    pltpu.make_async_copy(out_hbm.at[pl.ds(0, NPAT * Q)], pat_v, bsem).wait()

    def grp(g, carry):
        v = idx_v[pl.ds(L * g, L)]
        for j in range(L // Q):
            q = ((v[Q * j] * 2 + v[Q * j + 1]) * 2 + v[Q * j + 2]) * 2 \
                + v[Q * j + 3]
            pltpu.async_copy(
                pat_v.at[pl.ds(Q * q, Q)],
                out_hbm.at[pl.ds(base + L * g + Q * j, Q)],
                sem)
        return carry
    lax.fori_loop(0, BPW // L, grp, 0)

    my_out = out_hbm.at[pl.ds(base, BPW)]
    pltpu.make_async_copy(my_out, my_out, sem).wait()


def kernel(harmful, emb_weight):
    idx = jnp.reshape(harmful.astype(jnp.int32), (NW, BPW))
    return _lookup(idx, emb_weight)


# speedup vs baseline: 2.6879x; 2.6879x over previous
import functools

import jax
import jax.numpy as jnp
from jax import lax
from jax.experimental import pallas as pl
from jax.experimental.pallas import tpu as pltpu
from jax.experimental.pallas import tpu_sc as plsc

B = 16384
D = 256
L = 16
NC = 2
NS = 16
NW = NC * NS
BPW = B // NW
Q = 4
NQ = BPW // Q
NPAT = 1 << Q

_mesh = plsc.VectorSubcoreMesh(core_axis_name="c", subcore_axis_name="s")


@functools.partial(
    pl.kernel,
    mesh=_mesh,
    out_type=jax.ShapeDtypeStruct((B, D), jnp.float32),
    scratch_types=[
        pltpu.VMEM((BPW + L,), jnp.int32),
        pltpu.VMEM((2, D), jnp.float32),
        pltpu.VMEM((NPAT * Q, D), jnp.float32),
        pltpu.SemaphoreType.DMA,
    ],
)
def _lookup(idx_hbm, table_hbm, out_hbm, idx_v, table_v, pat_v, sem):
    wid = lax.axis_index("s") * NC + lax.axis_index("c")
    base = wid * BPW
    pltpu.sync_copy(table_hbm, table_v)
    pltpu.sync_copy(idx_hbm.at[wid], idx_v.at[pl.ds(0, BPW)])
    w0 = [table_v[0, pl.ds(c * L, L)] for c in range(D // L)]
    w1 = [table_v[1, pl.ds(c * L, L)] for c in range(D // L)]

    def build(k, carry):
        h = k & (Q - 1)
        p = k >> 2
        bit = lax.shift_right_logical(p, (Q - 1) - h) & 1
        take1 = bit != 0
        for c in range(D // L):
            pat_v[k, pl.ds(c * L, L)] = jnp.where(take1, w1[c], w0[c])
        return carry
    lax.fori_loop(0, NPAT * Q, build, 0)

    def grp(g, carry):
        v = idx_v[pl.ds(L * g, L)]
        for j in range(L // Q):
            q = ((v[Q * j] * 2 + v[Q * j + 1]) * 2 + v[Q * j + 2]) * 2 \
                + v[Q * j + 3]
            pltpu.async_copy(
                pat_v.at[pl.ds(Q * q, Q)],
                out_hbm.at[pl.ds(base + L * g + Q * j, Q)],
                sem)
        return carry
    lax.fori_loop(0, BPW // L, grp, 0)

    my_out = out_hbm.at[pl.ds(base, BPW)]
    pltpu.make_async_copy(my_out, my_out, sem).wait()


def kernel(harmful, emb_weight):
    idx = jnp.reshape(harmful.astype(jnp.int32), (NW, BPW))
    return _lookup(idx, emb_weight)
